# Initial kernel scaffold; baseline (speedup 1.0000x reference)
import functools

import jax
import jax.numpy as jnp
from jax import lax
from jax.experimental import pallas as pl
from jax.experimental.pallas import tpu as pltpu
from jax.experimental.pallas import tpu_sc as plsc

MAX_Z = 86
FEAT = 128

_NC = 2
_NS = 16
_NW = _NC * _NS
_CH = 128
_NCH = 25
_BPW = _CH * _NCH
_BPAD = _NW * _BPW


def _fuse_body(ec_ref, mw_ref, zt_ref, out_ref):
    ec = ec_ref[...]
    mw = mw_ref[...]
    out_ref[...] = lax.dot_general(
        ec, mw, (((1,), (1,)), ((), ())),
        preferred_element_type=jnp.float32,
    ) + zt_ref[...]


def _build_fused_table(elec_config, m_weight, z_table):
    return pl.pallas_call(
        _fuse_body,
        out_shape=jax.ShapeDtypeStruct((MAX_Z, FEAT), jnp.float32),
    )(elec_config[:MAX_Z], m_weight, z_table)


def _sc_gather(table, idx2d):
    mesh = plsc.VectorSubcoreMesh(core_axis_name="c", subcore_axis_name="s")

    @functools.partial(
        pl.kernel,
        mesh=mesh,
        out_type=jax.ShapeDtypeStruct((_BPAD, FEAT), jnp.float32),
        scratch_types=[
            pltpu.VMEM((_NCH, _CH), jnp.int32),
            pltpu.VMEM((_CH, FEAT), jnp.float32),
            pltpu.SemaphoreType.DMA,
        ],
    )
    def k(table_hbm, idx_hbm, out_hbm, idx_v, rows_v, sem):
        wid = lax.axis_index("s") * _NC + lax.axis_index("c")
        base = wid * _BPW
        pltpu.sync_copy(idx_hbm.at[pl.ds(wid * _NCH, _NCH)], idx_v)

        def body(j, carry):
            pltpu.async_copy(table_hbm.at[idx_v.at[j]], rows_v, sem).wait()
            pltpu.sync_copy(rows_v, out_hbm.at[pl.ds(base + j * _CH, _CH)])
            return carry

        lax.fori_loop(0, _NCH, body, 0)

    return k(table, idx2d)


def kernel(z, elec_config, m_weight, z_table):
    fused = _build_fused_table(elec_config, m_weight, z_table)
    zi = z.astype(jnp.int32)
    n = zi.shape[0]
    zi_pad = jnp.zeros((_BPAD,), jnp.int32).at[:n].set(zi)
    out = _sc_gather(fused, zi_pad.reshape(_NW * _NCH, _CH))
    return out[:n]


# speedup vs baseline: 1.4575x; 1.4575x over previous
import functools

import jax
import jax.numpy as jnp
from jax import lax
from jax.experimental import pallas as pl
from jax.experimental.pallas import tpu as pltpu
from jax.experimental.pallas import tpu_sc as plsc

MAX_Z = 86
FEAT = 128

_NC = 2
_NS = 16
_NW = _NC * _NS
_CH = 128
_NCH = 25
_BPW = _CH * _NCH
_BPAD = _NW * _BPW


def _fuse_body(ec_ref, mw_ref, zt_ref, out_ref):
    ec = ec_ref[...]
    mw = mw_ref[...]
    out_ref[...] = lax.dot_general(
        ec, mw, (((1,), (1,)), ((), ())),
        preferred_element_type=jnp.float32,
    ) + zt_ref[...]


def _build_fused_table(elec_config, m_weight, z_table):
    return pl.pallas_call(
        _fuse_body,
        out_shape=jax.ShapeDtypeStruct((MAX_Z, FEAT), jnp.float32),
    )(elec_config[:MAX_Z], m_weight, z_table)


def _sc_gather(table, idx2d):
    mesh = plsc.VectorSubcoreMesh(core_axis_name="c", subcore_axis_name="s")

    @functools.partial(
        pl.kernel,
        mesh=mesh,
        out_type=jax.ShapeDtypeStruct((_BPAD, FEAT), jnp.float32),
        scratch_types=[
            pltpu.VMEM((_NCH + 7, _CH), jnp.int32),
            pltpu.VMEM((_CH, FEAT), jnp.float32),
            pltpu.SemaphoreType.DMA,
        ],
    )
    def k(table_hbm, idx_hbm, out_hbm, idx_v, rows_v, sem):
        wid = lax.axis_index("s") * _NC + lax.axis_index("c")
        base = pl.multiple_of(wid * _BPW, 8)
        start = wid * _NCH
        off = lax.rem(start, 8)
        aligned = pl.multiple_of(start - off, 8)
        pltpu.sync_copy(idx_hbm.at[pl.ds(aligned, _NCH + 7)], idx_v)

        def body(j, carry):
            pltpu.async_copy(table_hbm.at[idx_v.at[off + j]], rows_v, sem).wait()
            pltpu.sync_copy(rows_v, out_hbm.at[pl.ds(base + j * _CH, _CH)])
            return carry

        lax.fori_loop(0, _NCH, body, 0)

    return k(table, idx2d)


def kernel(z, elec_config, m_weight, z_table):
    fused = _build_fused_table(elec_config, m_weight, z_table)
    zi = z.astype(jnp.int32)
    n = zi.shape[0]
    zi_pad = jnp.zeros((_BPAD,), jnp.int32).at[:n].set(zi)
    out = _sc_gather(fused, zi_pad.reshape(_NW * _NCH, _CH))
    return out[:n]


# speedup vs baseline: 1.5065x; 1.0336x over previous
import functools

import jax
import jax.numpy as jnp
from jax import lax
from jax.experimental import pallas as pl
from jax.experimental.pallas import tpu as pltpu
from jax.experimental.pallas import tpu_sc as plsc

MAX_Z = 86
FEAT = 128

_NC = 2
_NS = 16
_NW = _NC * _NS
_CH = 128
_NCH = 25
_BPW = _CH * _NCH
_BPAD = _NW * _BPW


def _fuse_body(ec_ref, mw_ref, zt_ref, out_ref):
    ec = ec_ref[...]
    mw = mw_ref[...]
    out_ref[...] = lax.dot_general(
        ec, mw, (((1,), (1,)), ((), ())),
        preferred_element_type=jnp.float32,
    ) + zt_ref[...]


def _build_fused_table(elec_config, m_weight, z_table):
    return pl.pallas_call(
        _fuse_body,
        out_shape=jax.ShapeDtypeStruct((MAX_Z, FEAT), jnp.float32),
    )(elec_config[:MAX_Z], m_weight, z_table)


def _sc_gather(table, idx2d):
    mesh = plsc.VectorSubcoreMesh(core_axis_name="c", subcore_axis_name="s")

    @functools.partial(
        pl.kernel,
        mesh=mesh,
        out_type=jax.ShapeDtypeStruct((_BPAD, FEAT), jnp.float32),
        scratch_types=[
            pltpu.VMEM((_NCH + 7, _CH), jnp.int32),
            pltpu.VMEM((_CH, FEAT), jnp.float32),
            pltpu.VMEM((_CH, FEAT), jnp.float32),
            pltpu.SemaphoreType.DMA,
            pltpu.SemaphoreType.DMA,
            pltpu.SemaphoreType.DMA,
            pltpu.SemaphoreType.DMA,
        ],
    )
    def k(table_hbm, idx_hbm, out_hbm, idx_v, buf_a, buf_b,
          gsem_a, gsem_b, ssem_a, ssem_b):
        wid = lax.axis_index("s") * _NC + lax.axis_index("c")
        base = pl.multiple_of(wid * _BPW, 8)
        start = wid * _NCH
        off = lax.rem(start, 8)
        aligned = pl.multiple_of(start - off, 8)
        pltpu.sync_copy(idx_hbm.at[pl.ds(aligned, _NCH + 7)], idx_v)

        def gather(j, buf, sem):
            return pltpu.make_async_copy(
                table_hbm.at[idx_v.at[off + j]], buf, sem)

        def store(j, buf, sem):
            return pltpu.make_async_copy(
                buf, out_hbm.at[pl.ds(base + j * _CH, _CH)], sem)

        gather(0, buf_a, gsem_a).start()

        def body(i, carry):
            j0 = 2 * i
            j1 = j0 + 1
            gather(j0, buf_a, gsem_a).wait()

            @pl.when(i > 0)
            def _():
                store(j0 - 1, buf_b, ssem_b).wait()

            gather(j1, buf_b, gsem_b).start()
            store(j0, buf_a, ssem_a).start()
            gather(j1, buf_b, gsem_b).wait()
            store(j0, buf_a, ssem_a).wait()
            gather(j0 + 2, buf_a, gsem_a).start()
            store(j1, buf_b, ssem_b).start()
            return carry

        lax.fori_loop(0, (_NCH - 1) // 2, body, 0)

        last = _NCH - 1
        gather(last, buf_a, gsem_a).wait()
        store(last - 1, buf_b, ssem_b).wait()
        store(last, buf_a, ssem_a).start()
        store(last, buf_a, ssem_a).wait()

    return k(table, idx2d)


def kernel(z, elec_config, m_weight, z_table):
    fused = _build_fused_table(elec_config, m_weight, z_table)
    zi = z.astype(jnp.int32)
    n = zi.shape[0]
    zi_pad = jnp.zeros((_BPAD,), jnp.int32).at[:n].set(zi)
    out = _sc_gather(fused, zi_pad.reshape(_NW * _NCH, _CH))
    return out[:n]
